# baseline (speedup 1.0000x reference)
import functools

import jax
import jax.numpy as jnp
from jax import lax
from jax.experimental import pallas as pl
from jax.experimental.pallas import tpu as pltpu
from jax.experimental.pallas import tpu_sc as plsc

LANES = 16
EPS = 1e-6


@functools.lru_cache(maxsize=None)
def _build_sc(ntok, hidden, chunk):
    info = plsc.get_sparse_core_info()
    nw = info.num_cores * info.num_subcores
    assert ntok % (nw * chunk) == 0
    tpw = ntok // nw
    nchunks = tpw // chunk
    dchunks = hidden // LANES
    mesh = plsc.VectorSubcoreMesh(core_axis_name="c", subcore_axis_name="s")

    @functools.partial(
        pl.kernel,
        out_type=(jax.ShapeDtypeStruct((ntok, hidden), jnp.float32),
                  jax.ShapeDtypeStruct((ntok, hidden), jnp.float32)),
        mesh=mesh,
        scratch_types=[
            pltpu.VMEM((2, chunk), jnp.int32),
            pltpu.VMEM((2, chunk), jnp.int32),
            pltpu.VMEM((chunk, hidden), jnp.float32),
            pltpu.VMEM((chunk, hidden), jnp.float32),
            pltpu.VMEM((chunk, hidden), jnp.float32),
            pltpu.VMEM((chunk, hidden), jnp.float32),
            pltpu.SemaphoreType.DMA,
            pltpu.SemaphoreType.DMA,
            pltpu.SemaphoreType.DMA,
            pltpu.SemaphoreType.DMA,
            pltpu.SemaphoreType.DMA,
            pltpu.SemaphoreType.DMA,
            pltpu.SemaphoreType.DMA,
            pltpu.SemaphoreType.DMA,
        ],
    )
    def sc_kernel(ids_hbm, pids_hbm, wtab_hbm, ptab_hbm, outw_hbm, outp_hbm,
                  widx_v, pidx_v, wbuf0, wbuf1, pbuf0, pbuf1,
                  sem_w0, sem_w1, sem_p0, sem_p1,
                  sem_ow0, sem_ow1, sem_op0, sem_op1):
        wid = lax.axis_index("s") * info.num_cores + lax.axis_index("c")
        base = wid * tpw
        wbufs = (wbuf0, wbuf1)
        pbufs = (pbuf0, pbuf1)
        wsems = (sem_w0, sem_w1)
        psems = (sem_p0, sem_p1)
        owsems = (sem_ow0, sem_ow1)
        opsems = (sem_op0, sem_op1)

        def idx_load(c, p):
            sl = pl.ds(base + c * chunk, chunk)
            pltpu.sync_copy(ids_hbm.at[sl], widx_v.at[p])
            pltpu.sync_copy(pids_hbm.at[sl], pidx_v.at[p])

        def start_gathers(p):
            gw = pltpu.async_copy(
                wtab_hbm.at[widx_v.at[p]], wbufs[p], wsems[p])
            gp = pltpu.async_copy(
                ptab_hbm.at[pidx_v.at[p]], pbufs[p], psems[p])
            return gw, gp

        def start_out(c, p):
            sl = pl.ds(base + c * chunk, chunk)
            dw = pltpu.async_copy(wbufs[p], outw_hbm.at[sl], owsems[p])
            dp = pltpu.async_copy(pbufs[p], outp_hbm.at[sl], opsems[p])
            return dw, dp

        idx_load(0, 0)
        gw, gp = start_gathers(0)

        out_dmas = [None, None]
        for c in range(nchunks):
            p = c % 2
            q = 1 - p
            gw.wait()
            gp.wait()
            if c + 1 < nchunks:
                if out_dmas[q] is not None:
                    for d in out_dmas[q]:
                        d.wait()
                    out_dmas[q] = None
                idx_load(c + 1, q)
            out_dmas[p] = start_out(c, p)
            if c + 1 < nchunks:
                gw, gp = start_gathers(q)
        for dmas in out_dmas:
            if dmas is not None:
                for d in dmas:
                    d.wait()

    return sc_kernel


def _tc_ln_kernel(tvocab, xw_ref, xp_ref, tid_ref, ttab_ref, w_ref, b_ref,
                  o_ref):
    x = xw_ref[...] + xp_ref[...]
    tid = tid_ref[...]
    trow = jnp.broadcast_to(ttab_ref[0][None, :], x.shape)
    for v in range(1, tvocab):
        trow = jnp.where(tid == v, ttab_ref[v][None, :], trow)
    x = x + trow
    mean = jnp.mean(x, axis=1, keepdims=True)
    var = jnp.mean(x * x, axis=1, keepdims=True) - mean * mean
    inv = lax.rsqrt(var + EPS)
    o_ref[...] = (x - mean) * (inv * w_ref[0][None, :]) + b_ref[0][None, :]


@functools.lru_cache(maxsize=None)
def _build_tc(ntok, hidden, tvocab, bt):
    ngrid = ntok // bt
    return pl.pallas_call(
        functools.partial(_tc_ln_kernel, tvocab),
        grid=(ngrid,),
        in_specs=[
            pl.BlockSpec((bt, hidden), lambda i: (i, 0)),
            pl.BlockSpec((bt, hidden), lambda i: (i, 0)),
            pl.BlockSpec((bt, 1), lambda i: (i, 0)),
            pl.BlockSpec((tvocab, hidden), lambda i: (0, 0)),
            pl.BlockSpec((1, hidden), lambda i: (0, 0)),
            pl.BlockSpec((1, hidden), lambda i: (0, 0)),
        ],
        out_specs=pl.BlockSpec((bt, hidden), lambda i: (i, 0)),
        out_shape=jax.ShapeDtypeStruct((ntok, hidden), jnp.float32),
    )


def kernel(input_ids, token_type_ids, position_ids, attention_mask,
           word_emb, pos_emb, type_emb, ln_weight, ln_bias):
    b, s = input_ids.shape
    ntok = b * s
    hidden = word_emb.shape[1]
    tvocab = type_emb.shape[0]
    ids = input_ids.reshape(ntok).astype(jnp.int32)
    pids = position_ids.reshape(ntok).astype(jnp.int32)
    bt = 256
    tids = token_type_ids.reshape(ntok, 1).astype(jnp.int32)
    lnw = ln_weight.reshape(1, hidden)
    lnb = ln_bias.reshape(1, hidden)
    nslab = 2
    slab = ntok // nslab
    sc = _build_sc(slab, hidden, 32)
    tc = _build_tc(slab, hidden, tvocab, bt)
    outs = []
    for k in range(nslab):
        sl = slice(k * slab, (k + 1) * slab)
        wrows, prows = sc(ids[sl], pids[sl], word_emb, pos_emb)
        outs.append(tc(wrows, prows, tids[sl], type_emb, lnw, lnb))
    out = jnp.concatenate(outs, axis=0)
    return out.reshape(b, s, hidden)


# speedup vs baseline: 1.2520x; 1.2520x over previous
import functools

import jax
import jax.numpy as jnp
from jax import lax
from jax.experimental import pallas as pl
from jax.experimental.pallas import tpu as pltpu
from jax.experimental.pallas import tpu_sc as plsc

LANES = 16
EPS = 1e-6


@functools.lru_cache(maxsize=None)
def _build_sc(ntok, hidden, chunk):
    info = plsc.get_sparse_core_info()
    nw = info.num_cores * info.num_subcores
    assert ntok % (nw * chunk) == 0
    tpw = ntok // nw
    nchunks = tpw // chunk
    dchunks = hidden // LANES
    mesh = plsc.VectorSubcoreMesh(core_axis_name="c", subcore_axis_name="s")

    @functools.partial(
        pl.kernel,
        out_type=(jax.ShapeDtypeStruct((ntok, hidden), jnp.float32),
                  jax.ShapeDtypeStruct((ntok, hidden), jnp.float32)),
        mesh=mesh,
        scratch_types=[
            pltpu.VMEM((2, chunk), jnp.int32),
            pltpu.VMEM((2, chunk), jnp.int32),
            pltpu.VMEM((chunk, hidden), jnp.float32),
            pltpu.VMEM((chunk, hidden), jnp.float32),
            pltpu.VMEM((chunk, hidden), jnp.float32),
            pltpu.VMEM((chunk, hidden), jnp.float32),
            pltpu.SemaphoreType.DMA,
            pltpu.SemaphoreType.DMA,
            pltpu.SemaphoreType.DMA,
            pltpu.SemaphoreType.DMA,
            pltpu.SemaphoreType.DMA,
            pltpu.SemaphoreType.DMA,
            pltpu.SemaphoreType.DMA,
            pltpu.SemaphoreType.DMA,
        ],
    )
    def sc_kernel(ids_hbm, pids_hbm, wtab_hbm, ptab_hbm, outw_hbm, outp_hbm,
                  widx_v, pidx_v, wbuf0, wbuf1, pbuf0, pbuf1,
                  sem_w0, sem_w1, sem_p0, sem_p1,
                  sem_ow0, sem_ow1, sem_op0, sem_op1):
        wid = lax.axis_index("s") * info.num_cores + lax.axis_index("c")
        base = wid * tpw
        wbufs = (wbuf0, wbuf1)
        pbufs = (pbuf0, pbuf1)
        wsems = (sem_w0, sem_w1)
        psems = (sem_p0, sem_p1)
        owsems = (sem_ow0, sem_ow1)
        opsems = (sem_op0, sem_op1)

        def idx_load(c, p):
            sl = pl.ds(base + c * chunk, chunk)
            pltpu.sync_copy(ids_hbm.at[sl], widx_v.at[p])
            pltpu.sync_copy(pids_hbm.at[sl], pidx_v.at[p])

        def start_gathers(p):
            gw = pltpu.async_copy(
                wtab_hbm.at[widx_v.at[p]], wbufs[p], wsems[p])
            gp = pltpu.async_copy(
                ptab_hbm.at[pidx_v.at[p]], pbufs[p], psems[p])
            return gw, gp

        def start_out(c, p):
            sl = pl.ds(base + c * chunk, chunk)
            dw = pltpu.async_copy(wbufs[p], outw_hbm.at[sl], owsems[p])
            dp = pltpu.async_copy(pbufs[p], outp_hbm.at[sl], opsems[p])
            return dw, dp

        idx_load(0, 0)
        gw, gp = start_gathers(0)

        out_dmas = [None, None]
        for c in range(nchunks):
            p = c % 2
            q = 1 - p
            gw.wait()
            gp.wait()
            if c + 1 < nchunks:
                if out_dmas[q] is not None:
                    for d in out_dmas[q]:
                        d.wait()
                    out_dmas[q] = None
                idx_load(c + 1, q)
            out_dmas[p] = start_out(c, p)
            if c + 1 < nchunks:
                gw, gp = start_gathers(q)
        for dmas in out_dmas:
            if dmas is not None:
                for d in dmas:
                    d.wait()

    return sc_kernel


def _tc_ln_kernel(tvocab, xw_ref, xp_ref, tid_ref, ttab_ref, w_ref, b_ref,
                  o_ref):
    x = xw_ref[...] + xp_ref[...]
    tid = tid_ref[...]
    trow = jnp.broadcast_to(ttab_ref[0][None, :], x.shape)
    for v in range(1, tvocab):
        trow = jnp.where(tid == v, ttab_ref[v][None, :], trow)
    x = x + trow
    mean = jnp.mean(x, axis=1, keepdims=True)
    var = jnp.mean(x * x, axis=1, keepdims=True) - mean * mean
    inv = lax.rsqrt(var + EPS)
    o_ref[...] = (x - mean) * (inv * w_ref[0][None, :]) + b_ref[0][None, :]


@functools.lru_cache(maxsize=None)
def _build_tc(ntok, hidden, tvocab, bt):
    ngrid = ntok // bt
    return pl.pallas_call(
        functools.partial(_tc_ln_kernel, tvocab),
        grid=(ngrid,),
        in_specs=[
            pl.BlockSpec((bt, hidden), lambda i: (i, 0)),
            pl.BlockSpec((bt, hidden), lambda i: (i, 0)),
            pl.BlockSpec((bt, 1), lambda i: (i, 0)),
            pl.BlockSpec((tvocab, hidden), lambda i: (0, 0)),
            pl.BlockSpec((1, hidden), lambda i: (0, 0)),
            pl.BlockSpec((1, hidden), lambda i: (0, 0)),
        ],
        out_specs=pl.BlockSpec((bt, hidden), lambda i: (i, 0)),
        out_shape=jax.ShapeDtypeStruct((ntok, hidden), jnp.float32),
    )


def kernel(input_ids, token_type_ids, position_ids, attention_mask,
           word_emb, pos_emb, type_emb, ln_weight, ln_bias):
    b, s = input_ids.shape
    ntok = b * s
    hidden = word_emb.shape[1]
    tvocab = type_emb.shape[0]
    ids = input_ids.reshape(ntok).astype(jnp.int32)
    pids = position_ids.reshape(ntok).astype(jnp.int32)
    bt = 512
    tids = token_type_ids.reshape(ntok, 1).astype(jnp.int32)
    sc = _build_sc(ntok, hidden, 32)
    wrows, prows = sc(ids, pids, word_emb, pos_emb)
    tc = _build_tc(ntok, hidden, tvocab, bt)
    out = tc(wrows, prows, tids, type_emb,
             ln_weight.reshape(1, hidden), ln_bias.reshape(1, hidden))
    return out.reshape(b, s, hidden)


# speedup vs baseline: 1.2702x; 1.0146x over previous
import functools

import jax
import jax.numpy as jnp
from jax import lax
from jax.experimental import pallas as pl
from jax.experimental.pallas import tpu as pltpu
from jax.experimental.pallas import tpu_sc as plsc

LANES = 16
EPS = 1e-6


@functools.lru_cache(maxsize=None)
def _build_sc(ntok, hidden, chunk):
    info = plsc.get_sparse_core_info()
    nw = info.num_cores * info.num_subcores
    assert ntok % (nw * chunk) == 0
    tpw = ntok // nw
    nchunks = tpw // chunk
    dchunks = hidden // LANES
    mesh = plsc.VectorSubcoreMesh(core_axis_name="c", subcore_axis_name="s")

    @functools.partial(
        pl.kernel,
        out_type=(jax.ShapeDtypeStruct((ntok, hidden), jnp.float32),
                  jax.ShapeDtypeStruct((ntok, hidden), jnp.float32)),
        mesh=mesh,
        scratch_types=[
            pltpu.VMEM((2, chunk), jnp.int32),
            pltpu.VMEM((2, chunk), jnp.int32),
            pltpu.VMEM((chunk, hidden), jnp.float32),
            pltpu.VMEM((chunk, hidden), jnp.float32),
            pltpu.VMEM((chunk, hidden), jnp.float32),
            pltpu.VMEM((chunk, hidden), jnp.float32),
            pltpu.SemaphoreType.DMA,
            pltpu.SemaphoreType.DMA,
            pltpu.SemaphoreType.DMA,
            pltpu.SemaphoreType.DMA,
            pltpu.SemaphoreType.DMA,
            pltpu.SemaphoreType.DMA,
            pltpu.SemaphoreType.DMA,
            pltpu.SemaphoreType.DMA,
        ],
    )
    def sc_kernel(ids_hbm, pids_hbm, wtab_hbm, ptab_hbm, outw_hbm, outp_hbm,
                  widx_v, pidx_v, wbuf0, wbuf1, pbuf0, pbuf1,
                  sem_w0, sem_w1, sem_p0, sem_p1,
                  sem_ow0, sem_ow1, sem_op0, sem_op1):
        wid = lax.axis_index("s") * info.num_cores + lax.axis_index("c")
        base = wid * tpw
        wbufs = (wbuf0, wbuf1)
        pbufs = (pbuf0, pbuf1)
        wsems = (sem_w0, sem_w1)
        psems = (sem_p0, sem_p1)
        owsems = (sem_ow0, sem_ow1)
        opsems = (sem_op0, sem_op1)

        def idx_load(c, p):
            sl = pl.ds(base + c * chunk, chunk)
            pltpu.sync_copy(ids_hbm.at[sl], widx_v.at[p])
            pltpu.sync_copy(pids_hbm.at[sl], pidx_v.at[p])

        def start_gathers(p):
            gw = pltpu.async_copy(
                wtab_hbm.at[widx_v.at[p]], wbufs[p], wsems[p])
            gp = pltpu.async_copy(
                ptab_hbm.at[pidx_v.at[p]], pbufs[p], psems[p])
            return gw, gp

        def start_out(c, p):
            sl = pl.ds(base + c * chunk, chunk)
            dw = pltpu.async_copy(wbufs[p], outw_hbm.at[sl], owsems[p])
            dp = pltpu.async_copy(pbufs[p], outp_hbm.at[sl], opsems[p])
            return dw, dp

        idx_load(0, 0)
        gw, gp = start_gathers(0)

        out_dmas = [None, None]
        for c in range(nchunks):
            p = c % 2
            q = 1 - p
            gw.wait()
            gp.wait()
            if c + 1 < nchunks:
                if out_dmas[q] is not None:
                    for d in out_dmas[q]:
                        d.wait()
                    out_dmas[q] = None
                idx_load(c + 1, q)
            out_dmas[p] = start_out(c, p)
            if c + 1 < nchunks:
                gw, gp = start_gathers(q)
        for dmas in out_dmas:
            if dmas is not None:
                for d in dmas:
                    d.wait()

    return sc_kernel


def _tc_ln_kernel(tvocab, xw_ref, xp_ref, tid_ref, ttab_ref, w_ref, b_ref,
                  o_ref):
    x = xw_ref[...] + xp_ref[...]
    tid = tid_ref[...]
    trow = jnp.broadcast_to(ttab_ref[0][None, :], x.shape)
    for v in range(1, tvocab):
        trow = jnp.where(tid == v, ttab_ref[v][None, :], trow)
    x = x + trow
    mean = jnp.mean(x, axis=1, keepdims=True)
    var = jnp.mean(x * x, axis=1, keepdims=True) - mean * mean
    inv = lax.rsqrt(var + EPS)
    o_ref[...] = (x - mean) * (inv * w_ref[0][None, :]) + b_ref[0][None, :]


@functools.lru_cache(maxsize=None)
def _build_tc(ntok, hidden, tvocab, bt):
    ngrid = ntok // bt
    return pl.pallas_call(
        functools.partial(_tc_ln_kernel, tvocab),
        grid=(ngrid,),
        in_specs=[
            pl.BlockSpec((bt, hidden), lambda i: (i, 0)),
            pl.BlockSpec((bt, hidden), lambda i: (i, 0)),
            pl.BlockSpec((bt, 1), lambda i: (i, 0)),
            pl.BlockSpec((tvocab, hidden), lambda i: (0, 0)),
            pl.BlockSpec((1, hidden), lambda i: (0, 0)),
            pl.BlockSpec((1, hidden), lambda i: (0, 0)),
        ],
        out_specs=pl.BlockSpec((bt, hidden), lambda i: (i, 0)),
        out_shape=jax.ShapeDtypeStruct((ntok, hidden), jnp.float32),
    )


def kernel(input_ids, token_type_ids, position_ids, attention_mask,
           word_emb, pos_emb, type_emb, ln_weight, ln_bias):
    b, s = input_ids.shape
    ntok = b * s
    hidden = word_emb.shape[1]
    tvocab = type_emb.shape[0]
    ids = input_ids.reshape(ntok).astype(jnp.int32)
    pids = position_ids.reshape(ntok).astype(jnp.int32)
    bt = 1024
    tids = token_type_ids.reshape(ntok, 1).astype(jnp.int32)
    sc = _build_sc(ntok, hidden, 32)
    wrows, prows = sc(ids, pids, word_emb, pos_emb)
    tc = _build_tc(ntok, hidden, tvocab, bt)
    out = tc(wrows, prows, tids, type_emb,
             ln_weight.reshape(1, hidden), ln_bias.reshape(1, hidden))
    return out.reshape(b, s, hidden)


# speedup vs baseline: 1.2819x; 1.0092x over previous
import functools

import jax
import jax.numpy as jnp
from jax import lax
from jax.experimental import pallas as pl
from jax.experimental.pallas import tpu as pltpu
from jax.experimental.pallas import tpu_sc as plsc

LANES = 16
EPS = 1e-6


@functools.lru_cache(maxsize=None)
def _build_sc(ntok, hidden, chunk):
    info = plsc.get_sparse_core_info()
    nw = info.num_cores * info.num_subcores
    assert ntok % (nw * chunk) == 0
    tpw = ntok // nw
    nchunks = tpw // chunk
    dchunks = hidden // LANES
    mesh = plsc.VectorSubcoreMesh(core_axis_name="c", subcore_axis_name="s")

    @functools.partial(
        pl.kernel,
        out_type=(jax.ShapeDtypeStruct((ntok, hidden), jnp.float32),
                  jax.ShapeDtypeStruct((ntok, hidden), jnp.float32)),
        mesh=mesh,
        scratch_types=[
            pltpu.VMEM((2, chunk), jnp.int32),
            pltpu.VMEM((2, chunk), jnp.int32),
            pltpu.VMEM((chunk, hidden), jnp.float32),
            pltpu.VMEM((chunk, hidden), jnp.float32),
            pltpu.VMEM((chunk, hidden), jnp.float32),
            pltpu.VMEM((chunk, hidden), jnp.float32),
            pltpu.SemaphoreType.DMA,
            pltpu.SemaphoreType.DMA,
            pltpu.SemaphoreType.DMA,
            pltpu.SemaphoreType.DMA,
            pltpu.SemaphoreType.DMA,
            pltpu.SemaphoreType.DMA,
            pltpu.SemaphoreType.DMA,
            pltpu.SemaphoreType.DMA,
        ],
    )
    def sc_kernel(ids_hbm, pids_hbm, wtab_hbm, ptab_hbm, outw_hbm, outp_hbm,
                  widx_v, pidx_v, wbuf0, wbuf1, pbuf0, pbuf1,
                  sem_w0, sem_w1, sem_p0, sem_p1,
                  sem_ow0, sem_ow1, sem_op0, sem_op1):
        wid = lax.axis_index("s") * info.num_cores + lax.axis_index("c")
        base = wid * tpw
        wbufs = (wbuf0, wbuf1)
        pbufs = (pbuf0, pbuf1)
        wsems = (sem_w0, sem_w1)
        psems = (sem_p0, sem_p1)
        owsems = (sem_ow0, sem_ow1)
        opsems = (sem_op0, sem_op1)

        def idx_load(c, p):
            sl = pl.ds(base + c * chunk, chunk)
            pltpu.sync_copy(ids_hbm.at[sl], widx_v.at[p])
            pltpu.sync_copy(pids_hbm.at[sl], pidx_v.at[p])

        def start_gathers(p):
            gw = pltpu.async_copy(
                wtab_hbm.at[widx_v.at[p]], wbufs[p], wsems[p])
            gp = pltpu.async_copy(
                ptab_hbm.at[pidx_v.at[p]], pbufs[p], psems[p])
            return gw, gp

        def start_out(c, p):
            sl = pl.ds(base + c * chunk, chunk)
            dw = pltpu.async_copy(wbufs[p], outw_hbm.at[sl], owsems[p])
            dp = pltpu.async_copy(pbufs[p], outp_hbm.at[sl], opsems[p])
            return dw, dp

        idx_load(0, 0)
        gw, gp = start_gathers(0)

        out_dmas = [None, None]
        for c in range(nchunks):
            p = c % 2
            q = 1 - p
            gw.wait()
            gp.wait()
            if c + 1 < nchunks:
                if out_dmas[q] is not None:
                    for d in out_dmas[q]:
                        d.wait()
                    out_dmas[q] = None
                idx_load(c + 1, q)
            out_dmas[p] = start_out(c, p)
            if c + 1 < nchunks:
                gw, gp = start_gathers(q)
        for dmas in out_dmas:
            if dmas is not None:
                for d in dmas:
                    d.wait()

    return sc_kernel


def _tc_ln_kernel(tvocab, xw_ref, xp_ref, tid_ref, ttab_ref, w_ref, b_ref,
                  o_ref):
    x = xw_ref[...] + xp_ref[...]
    tid = tid_ref[...]
    trow = jnp.broadcast_to(ttab_ref[0][None, :], x.shape)
    for v in range(1, tvocab):
        trow = jnp.where(tid == v, ttab_ref[v][None, :], trow)
    x = x + trow
    mean = jnp.mean(x, axis=1, keepdims=True)
    var = jnp.mean(x * x, axis=1, keepdims=True) - mean * mean
    inv = lax.rsqrt(var + EPS)
    o_ref[...] = (x - mean) * (inv * w_ref[0][None, :]) + b_ref[0][None, :]


@functools.lru_cache(maxsize=None)
def _build_tc(ntok, hidden, tvocab, bt):
    ngrid = ntok // bt
    return pl.pallas_call(
        functools.partial(_tc_ln_kernel, tvocab),
        grid=(ngrid,),
        in_specs=[
            pl.BlockSpec((bt, hidden), lambda i: (i, 0)),
            pl.BlockSpec((bt, hidden), lambda i: (i, 0)),
            pl.BlockSpec((bt, 1), lambda i: (i, 0)),
            pl.BlockSpec((tvocab, hidden), lambda i: (0, 0)),
            pl.BlockSpec((1, hidden), lambda i: (0, 0)),
            pl.BlockSpec((1, hidden), lambda i: (0, 0)),
        ],
        out_specs=pl.BlockSpec((bt, hidden), lambda i: (i, 0)),
        out_shape=jax.ShapeDtypeStruct((ntok, hidden), jnp.float32),
    )


def kernel(input_ids, token_type_ids, position_ids, attention_mask,
           word_emb, pos_emb, type_emb, ln_weight, ln_bias):
    b, s = input_ids.shape
    ntok = b * s
    hidden = word_emb.shape[1]
    tvocab = type_emb.shape[0]
    ids = input_ids.reshape(ntok).astype(jnp.int32)
    pids = position_ids.reshape(ntok).astype(jnp.int32)
    bt = 2048
    tids = token_type_ids.reshape(ntok, 1).astype(jnp.int32)
    sc = _build_sc(ntok, hidden, 32)
    wrows, prows = sc(ids, pids, word_emb, pos_emb)
    tc = _build_tc(ntok, hidden, tvocab, bt)
    out = tc(wrows, prows, tids, type_emb,
             ln_weight.reshape(1, hidden), ln_bias.reshape(1, hidden))
    return out.reshape(b, s, hidden)


# speedup vs baseline: 1.2845x; 1.0020x over previous
import functools

import jax
import jax.numpy as jnp
from jax import lax
from jax.experimental import pallas as pl
from jax.experimental.pallas import tpu as pltpu
from jax.experimental.pallas import tpu_sc as plsc

LANES = 16
EPS = 1e-6


@functools.lru_cache(maxsize=None)
def _build_sc(ntok, hidden, chunk):
    info = plsc.get_sparse_core_info()
    nw = info.num_cores * info.num_subcores
    assert ntok % (nw * chunk) == 0
    tpw = ntok // nw
    nchunks = tpw // chunk
    mesh = plsc.VectorSubcoreMesh(core_axis_name="c", subcore_axis_name="s")

    @functools.partial(
        pl.kernel,
        out_type=(jax.ShapeDtypeStruct((ntok, hidden), jnp.float32),
                  jax.ShapeDtypeStruct((ntok, hidden), jnp.float32)),
        mesh=mesh,
        scratch_types=[
            pltpu.VMEM((2, chunk), jnp.int32),
            pltpu.VMEM((2, chunk), jnp.int32),
            pltpu.VMEM((chunk, hidden), jnp.float32),
            pltpu.VMEM((chunk, hidden), jnp.float32),
            pltpu.VMEM((chunk, hidden), jnp.float32),
            pltpu.VMEM((chunk, hidden), jnp.float32),
            pltpu.SemaphoreType.DMA,
            pltpu.SemaphoreType.DMA,
            pltpu.SemaphoreType.DMA,
            pltpu.SemaphoreType.DMA,
            pltpu.SemaphoreType.DMA,
            pltpu.SemaphoreType.DMA,
            pltpu.SemaphoreType.DMA,
            pltpu.SemaphoreType.DMA,
        ],
    )
    def sc_kernel(ids_hbm, pids_hbm, wtab_hbm, ptab_hbm, outw_hbm, outp_hbm,
                  widx_v, pidx_v, wbuf0, wbuf1, pbuf0, pbuf1,
                  sem_w0, sem_w1, sem_p0, sem_p1,
                  sem_ow0, sem_ow1, sem_op0, sem_op1):
        wid = lax.axis_index("s") * info.num_cores + lax.axis_index("c")
        base = wid * tpw
        wbufs = (wbuf0, wbuf1)
        pbufs = (pbuf0, pbuf1)
        wsems = (sem_w0, sem_w1)
        psems = (sem_p0, sem_p1)
        owsems = (sem_ow0, sem_ow1)
        opsems = (sem_op0, sem_op1)

        def idx_load(c, p):
            sl = pl.ds(base + c * chunk, chunk)
            pltpu.sync_copy(ids_hbm.at[sl], widx_v.at[p])
            pltpu.sync_copy(pids_hbm.at[sl], pidx_v.at[p])

        def start_gathers(p):
            gw = pltpu.async_copy(
                wtab_hbm.at[widx_v.at[p]], wbufs[p], wsems[p])
            gp = pltpu.async_copy(
                ptab_hbm.at[pidx_v.at[p]], pbufs[p], psems[p])
            return gw, gp

        def start_out(c, p):
            sl = pl.ds(base + c * chunk, chunk)
            dw = pltpu.async_copy(wbufs[p], outw_hbm.at[sl], owsems[p])
            dp = pltpu.async_copy(pbufs[p], outp_hbm.at[sl], opsems[p])
            return dw, dp

        idx_load(0, 0)
        gw, gp = start_gathers(0)

        out_dmas = [None, None]
        for c in range(nchunks):
            p = c % 2
            q = 1 - p
            gw.wait()
            gp.wait()
            if c + 1 < nchunks:
                if out_dmas[q] is not None:
                    for d in out_dmas[q]:
                        d.wait()
                    out_dmas[q] = None
                idx_load(c + 1, q)
            out_dmas[p] = start_out(c, p)
            if c + 1 < nchunks:
                gw, gp = start_gathers(q)
        for dmas in out_dmas:
            if dmas is not None:
                for d in dmas:
                    d.wait()

    return sc_kernel


def _tc_ln_kernel(tvocab, xw_ref, xp_ref, tid_ref, ttab_ref, w_ref, b_ref,
                  o_ref):
    x = xw_ref[...] + xp_ref[...]
    tid = tid_ref[...]
    trow = jnp.broadcast_to(ttab_ref[0][None, :], x.shape)
    for v in range(1, tvocab):
        trow = jnp.where(tid == v, ttab_ref[v][None, :], trow)
    x = x + trow
    mean = jnp.mean(x, axis=1, keepdims=True)
    var = jnp.mean(x * x, axis=1, keepdims=True) - mean * mean
    inv = lax.rsqrt(var + EPS)
    o_ref[...] = (x - mean) * (inv * w_ref[0][None, :]) + b_ref[0][None, :]


@functools.lru_cache(maxsize=None)
def _build_tc(ntok, hidden, tvocab, bt):
    ngrid = ntok // bt
    return pl.pallas_call(
        functools.partial(_tc_ln_kernel, tvocab),
        grid=(ngrid,),
        in_specs=[
            pl.BlockSpec((bt, hidden), lambda i: (i, 0)),
            pl.BlockSpec((bt, hidden), lambda i: (i, 0)),
            pl.BlockSpec((bt, 1), lambda i: (i, 0)),
            pl.BlockSpec((tvocab, hidden), lambda i: (0, 0)),
            pl.BlockSpec((1, hidden), lambda i: (0, 0)),
            pl.BlockSpec((1, hidden), lambda i: (0, 0)),
        ],
        out_specs=pl.BlockSpec((bt, hidden), lambda i: (i, 0)),
        out_shape=jax.ShapeDtypeStruct((ntok, hidden), jnp.float32),
    )


def kernel(input_ids, token_type_ids, position_ids, attention_mask,
           word_emb, pos_emb, type_emb, ln_weight, ln_bias):
    b, s = input_ids.shape
    ntok = b * s
    hidden = word_emb.shape[1]
    tvocab = type_emb.shape[0]
    ids = input_ids.reshape(ntok).astype(jnp.int32)
    pids = position_ids.reshape(ntok).astype(jnp.int32)
    bt = 2048
    tids = token_type_ids.reshape(ntok, 1).astype(jnp.int32)
    sc = _build_sc(ntok, hidden, 32)
    wrows, prows = sc(ids, pids, word_emb, pos_emb)
    tc = _build_tc(ntok, hidden, tvocab, bt)
    out = tc(wrows, prows, tids, type_emb,
             ln_weight.reshape(1, hidden), ln_bias.reshape(1, hidden))
    return out.reshape(b, s, hidden)
